# baseline (speedup 1.0000x reference)
import functools

import jax
import jax.numpy as jnp
from jax import lax
from jax.experimental import pallas as pl
from jax.experimental.pallas import tpu as pltpu
from jax.experimental.pallas import tpu_sc as plsc

_N = 10000
_E = 320000
_NC = 2
_NS = 16
_IDXW = 64
_NROW = 5120
_FS_ROWS = _NROW // _NS
_ES_ROWS = _NROW // _NC // _NS
_ACCR = 10112
_WPS = _ACCR // _NS
_D = 128

_mesh = plsc.VectorSubcoreMesh(core_axis_name="c", subcore_axis_name="s")


def _make_agg(feature_split):
    nrows = _FS_ROWS if feature_split else _ES_ROWS
    ichunk = 16

    @functools.partial(
        pl.kernel,
        out_type=jax.ShapeDtypeStruct((_NC, _ACCR, _D), jnp.float32),
        mesh=_mesh,
        scratch_types=[
            pltpu.VMEM((ichunk, _IDXW), jnp.int32),
            pltpu.VMEM((ichunk, _IDXW), jnp.int32),
            pltpu.VMEM((_IDXW, _D), jnp.float32),
            pltpu.VMEM((_IDXW, _D), jnp.float32),
            pltpu.VMEM_SHARED((_ACCR, _D), jnp.float32),
            pltpu.SemaphoreType.DMA,
            pltpu.SemaphoreType.DMA,
        ],
    )
    def agg(src3, dst2, table, zrows, out,
            src_v, dst_v, rows_a, rows_b, acc, sem_a, sem_b):
        c = lax.axis_index("c")
        w = lax.axis_index("s")
        pltpu.sync_copy(zrows, acc.at[pl.ds(w * _WPS, _WPS)])
        if feature_split:
            start = _FS_ROWS * w
            plane = c
        else:
            start = (_NROW // _NC) * c + _ES_ROWS * w
            plane = 0
        plsc.subcore_barrier()

        bufs = (rows_a, rows_b)
        sems = (sem_a, sem_b)

        def outer(k, carry):
            base = start + k * ichunk
            pltpu.sync_copy(src3.at[plane, pl.ds(base, ichunk)], src_v)
            pltpu.sync_copy(dst2.at[pl.ds(base, ichunk)], dst_v)
            descs = [None, None]
            descs[0] = pltpu.async_copy(table.at[src_v.at[0]], bufs[0],
                                        sems[0])
            for j in range(ichunk):
                p = j % 2
                descs[p].wait()
                if j + 1 < ichunk:
                    q = (j + 1) % 2
                    descs[q] = pltpu.async_copy(
                        table.at[src_v.at[j + 1]], bufs[q], sems[q])
                pltpu.sync_copy(bufs[p], acc.at[dst_v.at[j]], add=True)
            return carry

        lax.fori_loop(0, nrows // ichunk, outer, 0)
        plsc.subcore_barrier()
        pltpu.sync_copy(acc.at[pl.ds(w * _WPS, _WPS)],
                        out.at[c, pl.ds(w * _WPS, _WPS)])

    return agg


@functools.partial(
    pl.kernel,
    out_type=jax.ShapeDtypeStruct((_NC, _ACCR, _D), jnp.float32),
    mesh=_mesh,
    scratch_types=[
        pltpu.VMEM((_ES_ROWS, _IDXW), jnp.int32),
        pltpu.VMEM((_IDXW, _D), jnp.float32),
        pltpu.VMEM_SHARED((_ACCR, _D), jnp.float32),
    ],
)
def _deg_kernel(dst2, zrows, ones, out, dst_v, ones_v, acc):
    c = lax.axis_index("c")
    w = lax.axis_index("s")
    pltpu.sync_copy(zrows, acc.at[pl.ds(w * _WPS, _WPS)])
    pltpu.sync_copy(ones, ones_v)
    start = (_NROW // _NC) * c + _ES_ROWS * w
    pltpu.sync_copy(dst2.at[pl.ds(start, _ES_ROWS)], dst_v)
    plsc.subcore_barrier()

    def body(j, carry):
        pltpu.sync_copy(ones_v, acc.at[dst_v.at[j]], add=True)
        return carry

    lax.fori_loop(0, _ES_ROWS, body, 0)
    plsc.subcore_barrier()
    pltpu.sync_copy(acc.at[pl.ds(w * _WPS, _WPS)],
                    out.at[c, pl.ds(w * _WPS, _WPS)])


_agg_fs = _make_agg(True)
_agg_es = _make_agg(False)

_R = 1000


def _mlp_body(x_ref, degp_ref, w1, b1, w2, b2, w3, b3, hs_ref, dinv_ref):
    h = jnp.maximum(jnp.dot(x_ref[...], w1[...],
                            preferred_element_type=jnp.float32) + b1[...], 0.0)
    h = jnp.maximum(jnp.dot(h, w2[...],
                            preferred_element_type=jnp.float32) + b2[...], 0.0)
    h = jnp.maximum(jnp.dot(h, w3[...],
                            preferred_element_type=jnp.float32) + b3[...], 0.0)
    deg = 1.0 + degp_ref[0, :, 0:16] + degp_ref[1, :, 0:16]
    dinv = lax.rsqrt(deg)
    hs_ref[...] = h * dinv[:, 0:1]
    dinv_ref[...] = dinv


def _gcn1_body(s_ref, hs_ref, dinv_ref, w, b, out_ref):
    dinv = dinv_ref[:, 0:1]
    t = (s_ref[0] + s_ref[1] + hs_ref[...]) * dinv
    o = jnp.maximum(jnp.dot(t, w[...],
                            preferred_element_type=jnp.float32) + b[...], 0.0)
    hs2 = o * dinv
    out_ref[0] = hs2[:, :128]
    out_ref[1] = hs2[:, 128:]


def _gcn2_body(s_ref, hs_ref, dinv_ref, w2, b2, w3, out_ref):
    dinv = dinv_ref[:, 0:1]
    t = jnp.concatenate([s_ref[0] + hs_ref[0], s_ref[1] + hs_ref[1]],
                        axis=1) * dinv
    o = jnp.maximum(jnp.dot(t, w2[...],
                            preferred_element_type=jnp.float32) + b2[...], 0.0)
    g = jnp.dot(o, w3[...], preferred_element_type=jnp.float32)
    hs3 = g * dinv
    out_ref[...] = jnp.concatenate(
        [hs3, jnp.zeros((_R, _D - 50), jnp.float32)], axis=1)


def _final_body(s_ref, hs_ref, dinv_ref, b3, out_ref):
    t = (s_ref[0] + s_ref[1] + hs_ref[...]) * dinv_ref[:, 0:1]
    logits = t[:, :50] + b3[...]
    m = jnp.max(logits, axis=1, keepdims=True)
    e = jnp.exp(logits - m)
    out_ref[...] = e / jnp.sum(e, axis=1, keepdims=True)


def _full(shape):
    return pl.BlockSpec(shape, lambda i: tuple(0 for _ in shape))


def _rows(shape):
    return pl.BlockSpec(shape, lambda i: (i,) + tuple(0 for _ in shape[1:]))


def _planes(shape):
    return pl.BlockSpec(shape, lambda i: (0, i, 0))


def kernel(x, edge_index, cnn1_W, cnn1_b, cnn2_W, cnn2_b, cnn3_W, cnn3_b,
           gcn1_W, gcn1_b, gcn2_W, gcn2_b, gcn3_W, gcn3_b):
    f32 = jnp.float32
    src = edge_index[0]
    dst = edge_index[1]
    npad = _NROW * _IDXW - _E
    src_p = jnp.concatenate([src, jnp.zeros((npad,), jnp.int32)])
    dst_p = jnp.concatenate([dst, jnp.full((npad,), _N, jnp.int32)])
    src3 = jnp.stack([src_p, src_p + _N]).reshape(_NC, _NROW, _IDXW)
    dst2 = dst_p.reshape(_NROW, _IDXW)

    zrows = jnp.zeros((_WPS, _D), f32)
    ones = jnp.ones((_IDXW, _D), f32)

    degp = _deg_kernel(dst2, zrows, ones)

    grid = (_N // _R,)
    hs1, dinv16 = pl.pallas_call(
        _mlp_body,
        grid=grid,
        in_specs=[
            _rows((_R, 3)), _planes((_NC, _R, _D)),
            _full((3, 32)), _full((1, 32)),
            _full((32, 64)), _full((1, 64)),
            _full((64, 128)), _full((1, 128)),
        ],
        out_specs=[_rows((_R, 128)), _rows((_R, 16))],
        out_shape=[jax.ShapeDtypeStruct((_N, 128), f32),
                   jax.ShapeDtypeStruct((_N, 16), f32)],
    )(x, degp, cnn1_W, cnn1_b.reshape(1, 32), cnn2_W, cnn2_b.reshape(1, 64),
      cnn3_W, cnn3_b.reshape(1, 128))

    s1 = _agg_es(src3, dst2, hs1, zrows)

    hs2 = pl.pallas_call(
        _gcn1_body,
        grid=grid,
        in_specs=[
            _planes((_NC, _R, _D)), _rows((_R, 128)), _rows((_R, 16)),
            _full((128, 256)), _full((1, 256)),
        ],
        out_specs=_planes((_NC, _R, 128)),
        out_shape=jax.ShapeDtypeStruct((_NC, _N, 128), f32),
    )(s1, hs1, dinv16, gcn1_W, gcn1_b.reshape(1, 256))

    s2 = _agg_fs(src3, dst2, hs2.reshape(_NC * _N, 128), zrows)

    hs3 = pl.pallas_call(
        _gcn2_body,
        grid=grid,
        in_specs=[
            _planes((_NC, _R, _D)), _planes((_NC, _R, 128)), _rows((_R, 16)),
            _full((256, 512)), _full((1, 512)), _full((512, 50)),
        ],
        out_specs=_rows((_R, _D)),
        out_shape=jax.ShapeDtypeStruct((_N, _D), f32),
    )(s2, hs2, dinv16, gcn2_W, gcn2_b.reshape(1, 512), gcn3_W)

    s3 = _agg_es(src3, dst2, hs3, zrows)

    out = pl.pallas_call(
        _final_body,
        grid=grid,
        in_specs=[
            _planes((_NC, _R, _D)), _rows((_R, _D)), _rows((_R, 16)),
            _full((1, 50)),
        ],
        out_specs=_rows((_R, 50)),
        out_shape=jax.ShapeDtypeStruct((_N, 50), f32),
    )(s3, hs3, dinv16, gcn3_b.reshape(1, 50))

    return out


# speedup vs baseline: 1.2308x; 1.2308x over previous
import functools

import jax
import jax.numpy as jnp
from jax import lax
from jax.experimental import pallas as pl
from jax.experimental.pallas import tpu as pltpu
from jax.experimental.pallas import tpu_sc as plsc

_N = 10000
_E = 320000
_NC = 2
_NS = 16
_IDXW = 128
_NROW = 2560
_FS_ROWS = _NROW // _NS
_ES_ROWS = _NROW // _NC // _NS
_ACCR = 10112
_WPS = _ACCR // _NS
_D = 128

_mesh = plsc.VectorSubcoreMesh(core_axis_name="c", subcore_axis_name="s")


def _make_agg(feature_split, dh=_D, tc_tiling=True):
    nrows = _FS_ROWS if feature_split else _ES_ROWS
    ichunk = 16
    cparams = None if tc_tiling else pltpu.CompilerParams(
        use_tc_tiling_on_sc=False)

    @functools.partial(
        pl.kernel,
        out_type=jax.ShapeDtypeStruct((_NC, _ACCR, dh), jnp.float32),
        mesh=_mesh,
        scratch_types=[
            pltpu.VMEM((ichunk, _IDXW), jnp.int32),
            pltpu.VMEM((ichunk, _IDXW), jnp.int32),
            pltpu.VMEM((_IDXW, dh), jnp.float32),
            pltpu.VMEM((_IDXW, dh), jnp.float32),
            pltpu.VMEM_SHARED((_ACCR, dh), jnp.float32),
            pltpu.SemaphoreType.DMA,
            pltpu.SemaphoreType.DMA,
        ],
        compiler_params=cparams,
    )
    def agg(src3, dst2, table, zrows, out,
            src_v, dst_v, rows_a, rows_b, acc, sem_a, sem_b):
        c = lax.axis_index("c")
        w = lax.axis_index("s")
        pltpu.sync_copy(zrows, acc.at[pl.ds(w * _WPS, _WPS)])
        if feature_split:
            start = _FS_ROWS * w
            plane = c
        else:
            start = (_NROW // _NC) * c + _ES_ROWS * w
            plane = 0
        plsc.subcore_barrier()

        bufs = (rows_a, rows_b)
        sems = (sem_a, sem_b)

        def outer(k, carry):
            base = start + k * ichunk
            pltpu.sync_copy(src3.at[plane, pl.ds(base, ichunk)], src_v)
            pltpu.sync_copy(dst2.at[pl.ds(base, ichunk)], dst_v)
            descs = [None, None]
            descs[0] = pltpu.async_copy(table.at[src_v.at[0]], bufs[0],
                                        sems[0])
            for j in range(ichunk):
                p = j % 2
                descs[p].wait()
                if j + 1 < ichunk:
                    q = (j + 1) % 2
                    descs[q] = pltpu.async_copy(
                        table.at[src_v.at[j + 1]], bufs[q], sems[q])
                pltpu.sync_copy(bufs[p], acc.at[dst_v.at[j]], add=True)
            return carry

        lax.fori_loop(0, nrows // ichunk, outer, 0)
        plsc.subcore_barrier()
        pltpu.sync_copy(acc.at[pl.ds(w * _WPS, _WPS)],
                        out.at[c, pl.ds(w * _WPS, _WPS)])

    return agg


_DEGW = 16

@functools.partial(
    pl.kernel,
    out_type=jax.ShapeDtypeStruct((_NC, _ACCR, _DEGW), jnp.float32),
    mesh=_mesh,
    scratch_types=[
        pltpu.VMEM((_ES_ROWS, _IDXW), jnp.int32),
        pltpu.VMEM((_IDXW, _DEGW), jnp.float32),
        pltpu.VMEM_SHARED((_ACCR, _DEGW), jnp.float32),
    ],
    compiler_params=pltpu.CompilerParams(use_tc_tiling_on_sc=False),
)
def _deg_kernel(dst2, zrows, ones, out, dst_v, ones_v, acc):
    c = lax.axis_index("c")
    w = lax.axis_index("s")
    pltpu.sync_copy(zrows, acc.at[pl.ds(w * _WPS, _WPS)])
    pltpu.sync_copy(ones, ones_v)
    start = (_NROW // _NC) * c + _ES_ROWS * w
    pltpu.sync_copy(dst2.at[pl.ds(start, _ES_ROWS)], dst_v)
    plsc.subcore_barrier()

    def body(j, carry):
        pltpu.sync_copy(ones_v, acc.at[dst_v.at[j]], add=True)
        return carry

    lax.fori_loop(0, _ES_ROWS, body, 0)
    plsc.subcore_barrier()
    pltpu.sync_copy(acc.at[pl.ds(w * _WPS, _WPS)],
                    out.at[c, pl.ds(w * _WPS, _WPS)])


_agg_fs = _make_agg(True)
_agg_es = _make_agg(False)
_agg_es64 = _make_agg(False, dh=64, tc_tiling=False)

_R = 1000


def _mlp_body(x_ref, degp_ref, w1, b1, w2, b2, w3, b3, hs_ref, dinv_ref):
    h = jnp.maximum(jnp.dot(x_ref[...], w1[...],
                            preferred_element_type=jnp.float32) + b1[...], 0.0)
    h = jnp.maximum(jnp.dot(h, w2[...],
                            preferred_element_type=jnp.float32) + b2[...], 0.0)
    h = jnp.maximum(jnp.dot(h, w3[...],
                            preferred_element_type=jnp.float32) + b3[...], 0.0)
    deg = 1.0 + degp_ref[0] + degp_ref[1]
    dinv = lax.rsqrt(deg)
    hs_ref[...] = h * dinv[:, 0:1]
    dinv_ref[...] = dinv


def _gcn1_body(s_ref, hs_ref, dinv_ref, w, b, out_ref):
    dinv = dinv_ref[:, 0:1]
    t = (s_ref[0] + s_ref[1] + hs_ref[...]) * dinv
    o = jnp.maximum(jnp.dot(t, w[...],
                            preferred_element_type=jnp.float32) + b[...], 0.0)
    hs2 = o * dinv
    out_ref[0] = hs2[:, :128]
    out_ref[1] = hs2[:, 128:]


def _gcn2_body(s_ref, hs_ref, dinv_ref, w2, b2, w3, out_ref):
    dinv = dinv_ref[:, 0:1]
    t = jnp.concatenate([s_ref[0] + hs_ref[0], s_ref[1] + hs_ref[1]],
                        axis=1) * dinv
    o = jnp.maximum(jnp.dot(t, w2[...],
                            preferred_element_type=jnp.float32) + b2[...], 0.0)
    g = jnp.dot(o, w3[...], preferred_element_type=jnp.float32)
    hs3 = g * dinv
    out_ref[...] = jnp.concatenate(
        [hs3, jnp.zeros((_R, 14), jnp.float32)], axis=1)


def _final_body(s_ref, hs_ref, dinv_ref, b3, out_ref):
    t = (s_ref[0] + s_ref[1] + hs_ref[...]) * dinv_ref[:, 0:1]
    logits = t[:, :50] + b3[...]
    m = jnp.max(logits, axis=1, keepdims=True)
    e = jnp.exp(logits - m)
    out_ref[...] = e / jnp.sum(e, axis=1, keepdims=True)


def _full(shape):
    return pl.BlockSpec(shape, lambda i: tuple(0 for _ in shape))


def _rows(shape):
    return pl.BlockSpec(shape, lambda i: (i,) + tuple(0 for _ in shape[1:]))


def _planes(shape):
    return pl.BlockSpec(shape, lambda i: (0, i, 0))


def kernel(x, edge_index, cnn1_W, cnn1_b, cnn2_W, cnn2_b, cnn3_W, cnn3_b,
           gcn1_W, gcn1_b, gcn2_W, gcn2_b, gcn3_W, gcn3_b):
    f32 = jnp.float32
    src = edge_index[0]
    dst = edge_index[1]
    npad = _NROW * _IDXW - _E
    src_p = jnp.concatenate([src, jnp.zeros((npad,), jnp.int32)])
    dst_p = jnp.concatenate([dst, jnp.full((npad,), _N, jnp.int32)])
    src3 = jnp.stack([src_p, src_p + _N]).reshape(_NC, _NROW, _IDXW)
    dst2 = dst_p.reshape(_NROW, _IDXW)

    zrows = jnp.zeros((_WPS, _D), f32)
    zrows64 = jnp.zeros((_WPS, 64), f32)
    zrows16 = jnp.zeros((_WPS, _DEGW), f32)
    ones = jnp.ones((_IDXW, _DEGW), f32)

    degp = _deg_kernel(dst2, zrows16, ones)

    grid = (_N // _R,)
    hs1, dinv16 = pl.pallas_call(
        _mlp_body,
        grid=grid,
        in_specs=[
            _rows((_R, 3)), _planes((_NC, _R, _DEGW)),
            _full((3, 32)), _full((1, 32)),
            _full((32, 64)), _full((1, 64)),
            _full((64, 128)), _full((1, 128)),
        ],
        out_specs=[_rows((_R, 128)), _rows((_R, 16))],
        out_shape=[jax.ShapeDtypeStruct((_N, 128), f32),
                   jax.ShapeDtypeStruct((_N, 16), f32)],
    )(x, degp, cnn1_W, cnn1_b.reshape(1, 32), cnn2_W, cnn2_b.reshape(1, 64),
      cnn3_W, cnn3_b.reshape(1, 128))

    s1 = _agg_es(src3, dst2, hs1, zrows)

    hs2 = pl.pallas_call(
        _gcn1_body,
        grid=grid,
        in_specs=[
            _planes((_NC, _R, _D)), _rows((_R, 128)), _rows((_R, 16)),
            _full((128, 256)), _full((1, 256)),
        ],
        out_specs=_planes((_NC, _R, 128)),
        out_shape=jax.ShapeDtypeStruct((_NC, _N, 128), f32),
    )(s1, hs1, dinv16, gcn1_W, gcn1_b.reshape(1, 256))

    s2 = _agg_fs(src3, dst2, hs2.reshape(_NC * _N, 128), zrows)

    hs3 = pl.pallas_call(
        _gcn2_body,
        grid=grid,
        in_specs=[
            _planes((_NC, _R, _D)), _planes((_NC, _R, 128)), _rows((_R, 16)),
            _full((256, 512)), _full((1, 512)), _full((512, 50)),
        ],
        out_specs=_rows((_R, 64)),
        out_shape=jax.ShapeDtypeStruct((_N, 64), f32),
    )(s2, hs2, dinv16, gcn2_W, gcn2_b.reshape(1, 512), gcn3_W)

    s3 = _agg_es64(src3, dst2, hs3, zrows64)

    out = pl.pallas_call(
        _final_body,
        grid=grid,
        in_specs=[
            _planes((_NC, _R, 64)), _rows((_R, 64)), _rows((_R, 16)),
            _full((1, 50)),
        ],
        out_specs=_rows((_R, 50)),
        out_shape=jax.ShapeDtypeStruct((_N, 50), f32),
    )(s3, hs3, dinv16, gcn3_b.reshape(1, 50))

    return out


# speedup vs baseline: 1.2331x; 1.0019x over previous
import functools

import jax
import jax.numpy as jnp
from jax import lax
from jax.experimental import pallas as pl
from jax.experimental.pallas import tpu as pltpu
from jax.experimental.pallas import tpu_sc as plsc

_N = 10000
_E = 320000
_NC = 2
_NS = 16
_IDXW = 128
_NROW = 2560
_FS_ROWS = _NROW // _NS
_ES_ROWS = _NROW // _NC // _NS
_ACCR = 10112
_WPS = _ACCR // _NS
_ACCB = 10240
_WPSB = _ACCB // _NS
_D = 128

_mesh = plsc.VectorSubcoreMesh(core_axis_name="c", subcore_axis_name="s")


def _make_agg(feature_split, dh=_D, tc_tiling=True, dtype=jnp.float32):
    nrows = _FS_ROWS if feature_split else _ES_ROWS
    ichunk = 16
    cparams = None if tc_tiling else pltpu.CompilerParams(
        use_tc_tiling_on_sc=False)
    accr = _ACCR if dtype == jnp.float32 else _ACCB
    wps = accr // _NS

    @functools.partial(
        pl.kernel,
        out_type=jax.ShapeDtypeStruct((_NC, accr, dh), dtype),
        mesh=_mesh,
        scratch_types=[
            pltpu.VMEM((ichunk, _IDXW), jnp.int32),
            pltpu.VMEM((ichunk, _IDXW), jnp.int32),
            pltpu.VMEM((_IDXW, dh), dtype),
            pltpu.VMEM((_IDXW, dh), dtype),
            pltpu.VMEM_SHARED((accr, dh), dtype),
            pltpu.SemaphoreType.DMA,
            pltpu.SemaphoreType.DMA,
            pltpu.SemaphoreType.DMA,
            pltpu.SemaphoreType.DMA,
        ],
        compiler_params=cparams,
    )
    def agg(src3, dst2, table, zrows, out,
            src_v, dst_v, rows_a, rows_b, acc, sem_a, sem_b, ssem_a, ssem_b):
        c = lax.axis_index("c")
        w = lax.axis_index("s")
        pltpu.sync_copy(zrows, acc.at[pl.ds(w * wps, wps)])
        if feature_split:
            start = _FS_ROWS * w
            plane = c
        else:
            start = (_NROW // _NC) * c + _ES_ROWS * w
            plane = 0
        plsc.subcore_barrier()

        bufs = (rows_a, rows_b)
        sems = (sem_a, sem_b)
        ssems = (ssem_a, ssem_b)

        def outer(k, carry):
            base = start + k * ichunk
            pltpu.sync_copy(src3.at[plane, pl.ds(base, ichunk)], src_v)
            pltpu.sync_copy(dst2.at[pl.ds(base, ichunk)], dst_v)
            descs = [None, None]
            sdescs = [None, None]
            descs[0] = pltpu.async_copy(table.at[src_v.at[0]], bufs[0],
                                        sems[0])
            for j in range(ichunk):
                p = j % 2
                descs[p].wait()
                sdescs[p] = pltpu.async_copy(bufs[p], acc.at[dst_v.at[j]],
                                             ssems[p], add=True)
                if j + 1 < ichunk:
                    q = (j + 1) % 2
                    if sdescs[q] is not None:
                        sdescs[q].wait()
                    descs[q] = pltpu.async_copy(
                        table.at[src_v.at[j + 1]], bufs[q], sems[q])
            sdescs[0].wait()
            sdescs[1].wait()
            return carry

        lax.fori_loop(0, nrows // ichunk, outer, 0)
        plsc.subcore_barrier()
        pltpu.sync_copy(acc.at[pl.ds(w * wps, wps)],
                        out.at[c, pl.ds(w * wps, wps)])

    return agg


_DEGW = 16

@functools.partial(
    pl.kernel,
    out_type=jax.ShapeDtypeStruct((_NC, _ACCR, _DEGW), jnp.float32),
    mesh=_mesh,
    scratch_types=[
        pltpu.VMEM((_ES_ROWS, _IDXW), jnp.int32),
        pltpu.VMEM((_IDXW, _DEGW), jnp.float32),
        pltpu.VMEM_SHARED((_ACCR, _DEGW), jnp.float32),
    ],
    compiler_params=pltpu.CompilerParams(use_tc_tiling_on_sc=False),
)
def _deg_kernel(dst2, zrows, ones, out, dst_v, ones_v, acc):
    c = lax.axis_index("c")
    w = lax.axis_index("s")
    pltpu.sync_copy(zrows, acc.at[pl.ds(w * _WPS, _WPS)])
    pltpu.sync_copy(ones, ones_v)
    start = (_NROW // _NC) * c + _ES_ROWS * w
    pltpu.sync_copy(dst2.at[pl.ds(start, _ES_ROWS)], dst_v)
    plsc.subcore_barrier()

    def body(j, carry):
        pltpu.sync_copy(ones_v, acc.at[dst_v.at[j]], add=True)
        return carry

    lax.fori_loop(0, _ES_ROWS, body, 0)
    plsc.subcore_barrier()
    pltpu.sync_copy(acc.at[pl.ds(w * _WPS, _WPS)],
                    out.at[c, pl.ds(w * _WPS, _WPS)])


_agg_fs = _make_agg(True)
_agg_es = _make_agg(False)
_agg_es64 = _make_agg(False, dh=64, tc_tiling=False)

_R = 1000


def _mlp_body(x_ref, degp_ref, w1, b1, w2, b2, w3, b3, hs_ref, dinv_ref):
    h = jnp.maximum(jnp.dot(x_ref[...], w1[...],
                            preferred_element_type=jnp.float32) + b1[...], 0.0)
    h = jnp.maximum(jnp.dot(h, w2[...],
                            preferred_element_type=jnp.float32) + b2[...], 0.0)
    h = jnp.maximum(jnp.dot(h, w3[...],
                            preferred_element_type=jnp.float32) + b3[...], 0.0)
    deg = 1.0 + degp_ref[0] + degp_ref[1]
    dinv = lax.rsqrt(deg)
    hs_ref[...] = h * dinv[:, 0:1]
    dinv_ref[...] = dinv


def _gcn1_body(s_ref, hs_ref, dinv_ref, w, b, out_ref):
    dinv = dinv_ref[:, 0:1]
    t = (s_ref[0] + s_ref[1] + hs_ref[...]) * dinv
    o = jnp.maximum(jnp.dot(t, w[...],
                            preferred_element_type=jnp.float32) + b[...], 0.0)
    hs2 = o * dinv
    out_ref[0] = hs2[:, :128]
    out_ref[1] = hs2[:, 128:]


def _gcn2_body(s_ref, hs_ref, dinv_ref, w2, b2, w3, out_ref):
    dinv = dinv_ref[:, 0:1]
    t = jnp.concatenate([s_ref[0] + hs_ref[0], s_ref[1] + hs_ref[1]],
                        axis=1) * dinv
    o = jnp.maximum(jnp.dot(t, w2[...],
                            preferred_element_type=jnp.float32) + b2[...], 0.0)
    g = jnp.dot(o, w3[...], preferred_element_type=jnp.float32)
    hs3 = g * dinv
    out_ref[...] = jnp.concatenate(
        [hs3, jnp.zeros((_R, 14), jnp.float32)], axis=1)


def _final_body(s_ref, hs_ref, dinv_ref, b3, out_ref):
    t = (s_ref[0] + s_ref[1] + hs_ref[...]) * dinv_ref[:, 0:1]
    logits = t[:, :50] + b3[...]
    m = jnp.max(logits, axis=1, keepdims=True)
    e = jnp.exp(logits - m)
    out_ref[...] = e / jnp.sum(e, axis=1, keepdims=True)


def _full(shape):
    return pl.BlockSpec(shape, lambda i: tuple(0 for _ in shape))


def _rows(shape):
    return pl.BlockSpec(shape, lambda i: (i,) + tuple(0 for _ in shape[1:]))


def _planes(shape):
    return pl.BlockSpec(shape, lambda i: (0, i, 0))


def kernel(x, edge_index, cnn1_W, cnn1_b, cnn2_W, cnn2_b, cnn3_W, cnn3_b,
           gcn1_W, gcn1_b, gcn2_W, gcn2_b, gcn3_W, gcn3_b):
    f32 = jnp.float32
    src = edge_index[0]
    dst = edge_index[1]
    npad = _NROW * _IDXW - _E
    src_p = jnp.concatenate([src, jnp.zeros((npad,), jnp.int32)])
    dst_p = jnp.concatenate([dst, jnp.full((npad,), _N, jnp.int32)])
    src3 = jnp.stack([src_p, src_p + _N]).reshape(_NC, _NROW, _IDXW)
    dst2 = dst_p.reshape(_NROW, _IDXW)

    zrows = jnp.zeros((_WPS, _D), f32)
    zrows64 = jnp.zeros((_WPS, 64), f32)
    zrows16 = jnp.zeros((_WPS, _DEGW), f32)
    ones = jnp.ones((_IDXW, _DEGW), f32)

    degp = _deg_kernel(dst2, zrows16, ones)

    grid = (_N // _R,)
    hs1, dinv16 = pl.pallas_call(
        _mlp_body,
        grid=grid,
        in_specs=[
            _rows((_R, 3)), _planes((_NC, _R, _DEGW)),
            _full((3, 32)), _full((1, 32)),
            _full((32, 64)), _full((1, 64)),
            _full((64, 128)), _full((1, 128)),
        ],
        out_specs=[_rows((_R, 128)), _rows((_R, 16))],
        out_shape=[jax.ShapeDtypeStruct((_N, 128), f32),
                   jax.ShapeDtypeStruct((_N, 16), f32)],
    )(x, degp, cnn1_W, cnn1_b.reshape(1, 32), cnn2_W, cnn2_b.reshape(1, 64),
      cnn3_W, cnn3_b.reshape(1, 128))

    s1 = _agg_es(src3, dst2, hs1, zrows)

    hs2 = pl.pallas_call(
        _gcn1_body,
        grid=grid,
        in_specs=[
            _planes((_NC, _R, _D)), _rows((_R, 128)), _rows((_R, 16)),
            _full((128, 256)), _full((1, 256)),
        ],
        out_specs=_planes((_NC, _R, 128)),
        out_shape=jax.ShapeDtypeStruct((_NC, _N, 128), f32),
    )(s1, hs1, dinv16, gcn1_W, gcn1_b.reshape(1, 256))

    s2 = _agg_fs(src3, dst2, hs2.reshape(_NC * _N, 128), zrows)

    hs3 = pl.pallas_call(
        _gcn2_body,
        grid=grid,
        in_specs=[
            _planes((_NC, _R, _D)), _planes((_NC, _R, 128)), _rows((_R, 16)),
            _full((256, 512)), _full((1, 512)), _full((512, 50)),
        ],
        out_specs=_rows((_R, 64)),
        out_shape=jax.ShapeDtypeStruct((_N, 64), f32),
    )(s2, hs2, dinv16, gcn2_W, gcn2_b.reshape(1, 512), gcn3_W)

    s3 = _agg_es64(src3, dst2, hs3, zrows64)

    out = pl.pallas_call(
        _final_body,
        grid=grid,
        in_specs=[
            _planes((_NC, _R, 64)), _rows((_R, 64)), _rows((_R, 16)),
            _full((1, 50)),
        ],
        out_specs=_rows((_R, 50)),
        out_shape=jax.ShapeDtypeStruct((_N, 50), f32),
    )(s3, hs3, dinv16, gcn3_b.reshape(1, 50))

    return out


# speedup vs baseline: 1.2989x; 1.0534x over previous
import functools

import jax
import jax.numpy as jnp
from jax import lax
from jax.experimental import pallas as pl
from jax.experimental.pallas import tpu as pltpu
from jax.experimental.pallas import tpu_sc as plsc

_N = 10000
_E = 320000
_NC = 2
_NS = 16
_IDXW = 128
_NROW = 2560
_FS_ROWS = _NROW // _NS
_ES_ROWS = _NROW // _NC // _NS
_ACCR = 10112
_WPS = _ACCR // _NS
_ACCB = 10240
_WPSB = _ACCB // _NS
_D = 128

_mesh = plsc.VectorSubcoreMesh(core_axis_name="c", subcore_axis_name="s")


def _make_agg(feature_split, dh=_D, tc_tiling=True, dtype=jnp.float32):
    nrows = _FS_ROWS if feature_split else _ES_ROWS
    ichunk = 16
    cparams = None if tc_tiling else pltpu.CompilerParams(
        use_tc_tiling_on_sc=False)
    accr = _ACCR if dtype == jnp.float32 else _ACCB
    wps = accr // _NS

    @functools.partial(
        pl.kernel,
        out_type=jax.ShapeDtypeStruct((_NC, accr, dh), dtype),
        mesh=_mesh,
        scratch_types=[
            pltpu.VMEM((ichunk, _IDXW), jnp.int32),
            pltpu.VMEM((ichunk, _IDXW), jnp.int32),
            pltpu.VMEM((_IDXW, dh), dtype),
            pltpu.VMEM((_IDXW, dh), dtype),
            pltpu.VMEM_SHARED((accr, dh), dtype),
            pltpu.SemaphoreType.DMA,
            pltpu.SemaphoreType.DMA,
            pltpu.SemaphoreType.DMA,
            pltpu.SemaphoreType.DMA,
        ],
        compiler_params=cparams,
    )
    def agg(src3, dst2, table, zrows, out,
            src_v, dst_v, rows_a, rows_b, acc, sem_a, sem_b, ssem_a, ssem_b):
        c = lax.axis_index("c")
        w = lax.axis_index("s")
        pltpu.sync_copy(zrows, acc.at[pl.ds(w * wps, wps)])
        if feature_split:
            start = _FS_ROWS * w
            plane = c
        else:
            start = None
            plane = 0
        plsc.subcore_barrier()

        bufs = (rows_a, rows_b)
        sems = (sem_a, sem_b)
        ssems = (ssem_a, ssem_b)

        def outer(k, carry):
            if feature_split:
                base = start + k * ichunk
            else:
                base = ((k * _NS + w) * _NC + c) * ichunk
            pltpu.sync_copy(src3.at[plane, pl.ds(base, ichunk)], src_v)
            pltpu.sync_copy(dst2.at[pl.ds(base, ichunk)], dst_v)
            descs = [None, None]
            sdescs = [None, None]
            descs[0] = pltpu.async_copy(table.at[src_v.at[0]], bufs[0],
                                        sems[0])
            for j in range(ichunk):
                p = j % 2
                descs[p].wait()
                sdescs[p] = pltpu.async_copy(bufs[p], acc.at[dst_v.at[j]],
                                             ssems[p], add=True)
                if j + 1 < ichunk:
                    q = (j + 1) % 2
                    if sdescs[q] is not None:
                        sdescs[q].wait()
                    descs[q] = pltpu.async_copy(
                        table.at[src_v.at[j + 1]], bufs[q], sems[q])
            sdescs[0].wait()
            sdescs[1].wait()
            return carry

        lax.fori_loop(0, nrows // ichunk, outer, 0)
        plsc.subcore_barrier()
        pltpu.sync_copy(acc.at[pl.ds(w * wps, wps)],
                        out.at[c, pl.ds(w * wps, wps)])

    return agg


_DEGW = 16

@functools.partial(
    pl.kernel,
    out_type=jax.ShapeDtypeStruct((_NC, _ACCR, _DEGW), jnp.float32),
    mesh=_mesh,
    scratch_types=[
        pltpu.VMEM((_ES_ROWS, _IDXW), jnp.int32),
        pltpu.VMEM((_IDXW, _DEGW), jnp.float32),
        pltpu.VMEM_SHARED((_ACCR, _DEGW), jnp.float32),
    ],
    compiler_params=pltpu.CompilerParams(use_tc_tiling_on_sc=False),
)
def _deg_kernel(dst2, zrows, ones, out, dst_v, ones_v, acc):
    c = lax.axis_index("c")
    w = lax.axis_index("s")
    pltpu.sync_copy(zrows, acc.at[pl.ds(w * _WPS, _WPS)])
    pltpu.sync_copy(ones, ones_v)
    start = (_NROW // _NC) * c + _ES_ROWS * w
    pltpu.sync_copy(dst2.at[pl.ds(start, _ES_ROWS)], dst_v)
    plsc.subcore_barrier()

    def body(j, carry):
        pltpu.sync_copy(ones_v, acc.at[dst_v.at[j]], add=True)
        return carry

    lax.fori_loop(0, _ES_ROWS, body, 0)
    plsc.subcore_barrier()
    pltpu.sync_copy(acc.at[pl.ds(w * _WPS, _WPS)],
                    out.at[c, pl.ds(w * _WPS, _WPS)])


_agg_fs = _make_agg(True)
_agg_es = _make_agg(False)
_agg_es64 = _make_agg(False, dh=64, tc_tiling=False)

_R = 1000


def _mlp_body(x_ref, degp_ref, w1, b1, w2, b2, w3, b3, hs_ref, dinv_ref):
    h = jnp.maximum(jnp.dot(x_ref[...], w1[...],
                            preferred_element_type=jnp.float32) + b1[...], 0.0)
    h = jnp.maximum(jnp.dot(h, w2[...],
                            preferred_element_type=jnp.float32) + b2[...], 0.0)
    h = jnp.maximum(jnp.dot(h, w3[...],
                            preferred_element_type=jnp.float32) + b3[...], 0.0)
    deg = 1.0 + degp_ref[0] + degp_ref[1]
    dinv = lax.rsqrt(deg)
    hs_ref[...] = h * dinv[:, 0:1]
    dinv_ref[...] = dinv


def _gcn1_body(s_ref, hs_ref, dinv_ref, w, b, out_ref):
    dinv = dinv_ref[:, 0:1]
    t = (s_ref[0] + s_ref[1] + hs_ref[...]) * dinv
    o = jnp.maximum(jnp.dot(t, w[...],
                            preferred_element_type=jnp.float32) + b[...], 0.0)
    hs2 = o * dinv
    out_ref[0] = hs2[:, :128]
    out_ref[1] = hs2[:, 128:]


def _gcn2_body(s_ref, hs_ref, dinv_ref, w2, b2, w3, out_ref):
    dinv = dinv_ref[:, 0:1]
    t = jnp.concatenate([s_ref[0] + hs_ref[0], s_ref[1] + hs_ref[1]],
                        axis=1) * dinv
    o = jnp.maximum(jnp.dot(t, w2[...],
                            preferred_element_type=jnp.float32) + b2[...], 0.0)
    g = jnp.dot(o, w3[...], preferred_element_type=jnp.float32)
    hs3 = g * dinv
    out_ref[...] = jnp.concatenate(
        [hs3, jnp.zeros((_R, 14), jnp.float32)], axis=1)


def _final_body(s_ref, hs_ref, dinv_ref, b3, out_ref):
    t = (s_ref[0] + s_ref[1] + hs_ref[...]) * dinv_ref[:, 0:1]
    logits = t[:, :50] + b3[...]
    m = jnp.max(logits, axis=1, keepdims=True)
    e = jnp.exp(logits - m)
    out_ref[...] = e / jnp.sum(e, axis=1, keepdims=True)


def _full(shape):
    return pl.BlockSpec(shape, lambda i: tuple(0 for _ in shape))


def _rows(shape):
    return pl.BlockSpec(shape, lambda i: (i,) + tuple(0 for _ in shape[1:]))


def _planes(shape):
    return pl.BlockSpec(shape, lambda i: (0, i, 0))


def kernel(x, edge_index, cnn1_W, cnn1_b, cnn2_W, cnn2_b, cnn3_W, cnn3_b,
           gcn1_W, gcn1_b, gcn2_W, gcn2_b, gcn3_W, gcn3_b):
    f32 = jnp.float32
    src = edge_index[0]
    dst = edge_index[1]
    npad = _NROW * _IDXW - _E
    src_p = jnp.concatenate([src, jnp.zeros((npad,), jnp.int32)])
    dst_p = jnp.concatenate([dst, jnp.full((npad,), _N, jnp.int32)])
    src3 = jnp.stack([src_p, src_p + _N]).reshape(_NC, _NROW, _IDXW)
    dst2 = dst_p.reshape(_NROW, _IDXW)

    zrows = jnp.zeros((_WPS, _D), f32)
    zrows64 = jnp.zeros((_WPS, 64), f32)
    zrows16 = jnp.zeros((_WPS, _DEGW), f32)
    ones = jnp.ones((_IDXW, _DEGW), f32)

    degp = _deg_kernel(dst2, zrows16, ones)

    grid = (_N // _R,)
    hs1, dinv16 = pl.pallas_call(
        _mlp_body,
        grid=grid,
        in_specs=[
            _rows((_R, 3)), _planes((_NC, _R, _DEGW)),
            _full((3, 32)), _full((1, 32)),
            _full((32, 64)), _full((1, 64)),
            _full((64, 128)), _full((1, 128)),
        ],
        out_specs=[_rows((_R, 128)), _rows((_R, 16))],
        out_shape=[jax.ShapeDtypeStruct((_N, 128), f32),
                   jax.ShapeDtypeStruct((_N, 16), f32)],
    )(x, degp, cnn1_W, cnn1_b.reshape(1, 32), cnn2_W, cnn2_b.reshape(1, 64),
      cnn3_W, cnn3_b.reshape(1, 128))

    s1 = _agg_es(src3, dst2, hs1, zrows)

    hs2 = pl.pallas_call(
        _gcn1_body,
        grid=grid,
        in_specs=[
            _planes((_NC, _R, _D)), _rows((_R, 128)), _rows((_R, 16)),
            _full((128, 256)), _full((1, 256)),
        ],
        out_specs=_planes((_NC, _R, 128)),
        out_shape=jax.ShapeDtypeStruct((_NC, _N, 128), f32),
    )(s1, hs1, dinv16, gcn1_W, gcn1_b.reshape(1, 256))

    s2 = _agg_fs(src3, dst2, hs2.reshape(_NC * _N, 128), zrows)

    hs3 = pl.pallas_call(
        _gcn2_body,
        grid=grid,
        in_specs=[
            _planes((_NC, _R, _D)), _planes((_NC, _R, 128)), _rows((_R, 16)),
            _full((256, 512)), _full((1, 512)), _full((512, 50)),
        ],
        out_specs=_rows((_R, 64)),
        out_shape=jax.ShapeDtypeStruct((_N, 64), f32),
    )(s2, hs2, dinv16, gcn2_W, gcn2_b.reshape(1, 512), gcn3_W)

    s3 = _agg_es64(src3, dst2, hs3, zrows64)

    out = pl.pallas_call(
        _final_body,
        grid=grid,
        in_specs=[
            _planes((_NC, _R, 64)), _rows((_R, 64)), _rows((_R, 16)),
            _full((1, 50)),
        ],
        out_specs=_rows((_R, 50)),
        out_shape=jax.ShapeDtypeStruct((_N, 50), f32),
    )(s3, hs3, dinv16, gcn3_b.reshape(1, 50))

    return out


# speedup vs baseline: 1.4023x; 1.0796x over previous
import functools

import jax
import jax.numpy as jnp
from jax import lax
from jax.experimental import pallas as pl
from jax.experimental.pallas import tpu as pltpu
from jax.experimental.pallas import tpu_sc as plsc

_N = 10000
_E = 320000
_NC = 2
_NS = 16
_IDXW = 128
_NROW = 2560
_FS_ROWS = _NROW // _NS
_ES_ROWS = _NROW // _NC // _NS
_ACCR = 10112
_WPS = _ACCR // _NS
_ACCB = 10240
_WPSB = _ACCB // _NS
_D = 128

_mesh = plsc.VectorSubcoreMesh(core_axis_name="c", subcore_axis_name="s")


def _make_agg(feature_split, dh=_D, tc_tiling=True, dtype=jnp.float32):
    nrows = _FS_ROWS if feature_split else _ES_ROWS
    ichunk = 16
    cparams = None if tc_tiling else pltpu.CompilerParams(
        use_tc_tiling_on_sc=False)
    accr = _ACCR if dtype == jnp.float32 else _ACCB
    wps = accr // _NS

    @functools.partial(
        pl.kernel,
        out_type=jax.ShapeDtypeStruct((_NC, accr, dh), dtype),
        mesh=_mesh,
        scratch_types=[
            pltpu.VMEM((ichunk, _IDXW), jnp.int32),
            pltpu.VMEM((ichunk, _IDXW), jnp.int32),
            pltpu.VMEM((_IDXW, dh), dtype),
            pltpu.VMEM((_IDXW, dh), dtype),
            pltpu.VMEM_SHARED((accr, dh), dtype),
            pltpu.SemaphoreType.DMA,
            pltpu.SemaphoreType.DMA,
            pltpu.SemaphoreType.DMA,
            pltpu.SemaphoreType.DMA,
        ],
        compiler_params=cparams,
    )
    def agg(src3, dst2, table, zrows, out,
            src_v, dst_v, rows_a, rows_b, acc, sem_a, sem_b, ssem_a, ssem_b):
        c = lax.axis_index("c")
        w = lax.axis_index("s")
        pltpu.sync_copy(zrows, acc.at[pl.ds(w * wps, wps)])
        if feature_split:
            start = _FS_ROWS * w
            plane = c
        else:
            start = None
            plane = 0
        plsc.subcore_barrier()

        bufs = (rows_a, rows_b)
        sems = (sem_a, sem_b)
        ssems = (ssem_a, ssem_b)

        def outer(k, carry):
            if feature_split:
                base = start + k * ichunk
            else:
                base = ((k * _NS + w) * _NC + c) * ichunk
            pltpu.sync_copy(src3.at[plane, pl.ds(base, ichunk)], src_v)
            pltpu.sync_copy(dst2.at[pl.ds(base, ichunk)], dst_v)
            descs = [None, None]
            sdescs = [None, None]
            descs[0] = pltpu.async_copy(table.at[src_v.at[0]], bufs[0],
                                        sems[0])
            for j in range(ichunk):
                p = j % 2
                descs[p].wait()
                sdescs[p] = pltpu.async_copy(bufs[p], acc.at[dst_v.at[j]],
                                             ssems[p], add=True)
                if j + 1 < ichunk:
                    q = (j + 1) % 2
                    if sdescs[q] is not None:
                        sdescs[q].wait()
                    descs[q] = pltpu.async_copy(
                        table.at[src_v.at[j + 1]], bufs[q], sems[q])
            sdescs[0].wait()
            sdescs[1].wait()
            return carry

        lax.fori_loop(0, nrows // ichunk, outer, 0)
        plsc.subcore_barrier()
        pltpu.sync_copy(acc.at[pl.ds(w * wps, wps)],
                        out.at[c, pl.ds(w * wps, wps)])

    return agg


_DEGW = 16

@functools.partial(
    pl.kernel,
    out_type=jax.ShapeDtypeStruct((_NC, _ACCR, _DEGW), jnp.float32),
    mesh=_mesh,
    scratch_types=[
        pltpu.VMEM((_ES_ROWS, _IDXW), jnp.int32),
        pltpu.VMEM((_IDXW, _DEGW), jnp.float32),
        pltpu.VMEM_SHARED((_ACCR, _DEGW), jnp.float32),
    ],
    compiler_params=pltpu.CompilerParams(use_tc_tiling_on_sc=False),
)
def _deg_kernel(dst2, zrows, ones, out, dst_v, ones_v, acc):
    c = lax.axis_index("c")
    w = lax.axis_index("s")
    pltpu.sync_copy(zrows, acc.at[pl.ds(w * _WPS, _WPS)])
    pltpu.sync_copy(ones, ones_v)
    start = (_NROW // _NC) * c + _ES_ROWS * w
    pltpu.sync_copy(dst2.at[pl.ds(start, _ES_ROWS)], dst_v)
    plsc.subcore_barrier()

    def body(j, carry):
        pltpu.sync_copy(ones_v, acc.at[dst_v.at[j]], add=True)
        return carry

    lax.fori_loop(0, _ES_ROWS, body, 0)
    plsc.subcore_barrier()
    pltpu.sync_copy(acc.at[pl.ds(w * _WPS, _WPS)],
                    out.at[c, pl.ds(w * _WPS, _WPS)])


_agg_fs = _make_agg(True)
_agg_es = _make_agg(False)
_agg_es64 = _make_agg(False, dh=64, tc_tiling=False)

_R = 1000


def _mlp_body(x_ref, degp_ref, w1, b1, w2, b2, w3, b3, hs_ref, dinv_ref):
    h = jnp.maximum(jnp.dot(x_ref[...], w1[...],
                            preferred_element_type=jnp.float32) + b1[...], 0.0)
    h = jnp.maximum(jnp.dot(h, w2[...],
                            preferred_element_type=jnp.float32) + b2[...], 0.0)
    h = jnp.maximum(jnp.dot(h, w3[...],
                            preferred_element_type=jnp.float32) + b3[...], 0.0)
    deg = 1.0 + degp_ref[0] + degp_ref[1]
    dinv = lax.rsqrt(deg)
    hs_ref[...] = h * dinv[:, 0:1]
    dinv_ref[...] = dinv


def _gcn1_body(s_ref, hs_ref, dinv_ref, w, b, out_ref):
    dinv = dinv_ref[:, 0:1]
    t = (s_ref[0] + s_ref[1] + hs_ref[...]) * dinv
    o = jnp.maximum(jnp.dot(t, w[...],
                            preferred_element_type=jnp.float32) + b[...], 0.0)
    hs2 = o * dinv
    out_ref[0] = hs2[:, :128]
    out_ref[1] = hs2[:, 128:]


def _gcn2_body(s_ref, hs_ref, dinv_ref, w2, b2, w3, out_ref):
    dinv = dinv_ref[:, 0:1]
    t = jnp.concatenate([s_ref[0] + hs_ref[0], s_ref[1] + hs_ref[1]],
                        axis=1) * dinv
    o = jnp.maximum(jnp.dot(t, w2[...],
                            preferred_element_type=jnp.float32) + b2[...], 0.0)
    g = jnp.dot(o, w3[...], preferred_element_type=jnp.float32)
    hs3 = g * dinv
    out_ref[...] = jnp.concatenate(
        [hs3, jnp.zeros((_R, 14), jnp.float32)], axis=1)


def _final_body(s_ref, hs_ref, dinv_ref, b3, out_ref):
    t = (s_ref[0] + s_ref[1] + hs_ref[...]) * dinv_ref[:, 0:1]
    logits = t[:, :50] + b3[...]
    m = jnp.max(logits, axis=1, keepdims=True)
    e = jnp.exp(logits - m)
    out_ref[...] = e / jnp.sum(e, axis=1, keepdims=True)


def _full(shape):
    return pl.BlockSpec(shape, lambda i: tuple(0 for _ in shape))


def _rows(shape):
    return pl.BlockSpec(shape, lambda i: (i,) + tuple(0 for _ in shape[1:]))


def _planes(shape):
    return pl.BlockSpec(shape, lambda i: (0, i, 0))


def kernel(x, edge_index, cnn1_W, cnn1_b, cnn2_W, cnn2_b, cnn3_W, cnn3_b,
           gcn1_W, gcn1_b, gcn2_W, gcn2_b, gcn3_W, gcn3_b):
    f32 = jnp.float32
    src = edge_index[0]
    dst = edge_index[1]
    npad = _NROW * _IDXW - _E
    src_p = jnp.concatenate([src, jnp.zeros((npad,), jnp.int32)])
    pad_dst = _N + jnp.arange(npad, dtype=jnp.int32) % (_ACCR - _N - 2)
    dst_p = jnp.concatenate([dst, pad_dst])
    src3 = jnp.stack([src_p, src_p + _N]).reshape(_NC, _NROW, _IDXW)
    dst2 = dst_p.reshape(_NROW, _IDXW)

    zrows = jnp.zeros((_WPS, _D), f32)
    zrows64 = jnp.zeros((_WPS, 64), f32)
    zrows16 = jnp.zeros((_WPS, _DEGW), f32)
    ones = jnp.ones((_IDXW, _DEGW), f32)

    degp = _deg_kernel(dst2, zrows16, ones)

    grid = (_N // _R,)
    hs1, dinv16 = pl.pallas_call(
        _mlp_body,
        grid=grid,
        in_specs=[
            _rows((_R, 3)), _planes((_NC, _R, _DEGW)),
            _full((3, 32)), _full((1, 32)),
            _full((32, 64)), _full((1, 64)),
            _full((64, 128)), _full((1, 128)),
        ],
        out_specs=[_rows((_R, 128)), _rows((_R, 16))],
        out_shape=[jax.ShapeDtypeStruct((_N, 128), f32),
                   jax.ShapeDtypeStruct((_N, 16), f32)],
    )(x, degp, cnn1_W, cnn1_b.reshape(1, 32), cnn2_W, cnn2_b.reshape(1, 64),
      cnn3_W, cnn3_b.reshape(1, 128))

    s1 = _agg_es(src3, dst2, hs1, zrows)

    hs2 = pl.pallas_call(
        _gcn1_body,
        grid=grid,
        in_specs=[
            _planes((_NC, _R, _D)), _rows((_R, 128)), _rows((_R, 16)),
            _full((128, 256)), _full((1, 256)),
        ],
        out_specs=_planes((_NC, _R, 128)),
        out_shape=jax.ShapeDtypeStruct((_NC, _N, 128), f32),
    )(s1, hs1, dinv16, gcn1_W, gcn1_b.reshape(1, 256))

    s2 = _agg_fs(src3, dst2, hs2.reshape(_NC * _N, 128), zrows)

    hs3 = pl.pallas_call(
        _gcn2_body,
        grid=grid,
        in_specs=[
            _planes((_NC, _R, _D)), _planes((_NC, _R, 128)), _rows((_R, 16)),
            _full((256, 512)), _full((1, 512)), _full((512, 50)),
        ],
        out_specs=_rows((_R, 64)),
        out_shape=jax.ShapeDtypeStruct((_N, 64), f32),
    )(s2, hs2, dinv16, gcn2_W, gcn2_b.reshape(1, 512), gcn3_W)

    s3 = _agg_es64(src3, dst2, hs3, zrows64)

    out = pl.pallas_call(
        _final_body,
        grid=grid,
        in_specs=[
            _planes((_NC, _R, 64)), _rows((_R, 64)), _rows((_R, 16)),
            _full((1, 50)),
        ],
        out_specs=_rows((_R, 50)),
        out_shape=jax.ShapeDtypeStruct((_N, 50), f32),
    )(s3, hs3, dinv16, gcn3_b.reshape(1, 50))

    return out


# speedup vs baseline: 1.6424x; 1.1712x over previous
import functools

import jax
import jax.numpy as jnp
from jax import lax
from jax.experimental import pallas as pl
from jax.experimental.pallas import tpu as pltpu
from jax.experimental.pallas import tpu_sc as plsc

_N = 10000
_E = 320000
_NC = 2
_NS = 16
_IDXW = 128
_NROW = 2560
_FS_ROWS = _NROW // _NS
_ES_ROWS = _NROW // _NC // _NS
_ACCR = 10112
_WPS = _ACCR // _NS
_ACCB = 10240
_WPSB = _ACCB // _NS
_D = 128

_mesh = plsc.VectorSubcoreMesh(core_axis_name="c", subcore_axis_name="s")


def _make_agg(feature_split, dh=_D, tc_tiling=True, dtype=jnp.float32):
    nrows = _FS_ROWS if feature_split else _ES_ROWS
    ichunk = 16
    cparams = None if tc_tiling else pltpu.CompilerParams(
        use_tc_tiling_on_sc=False)
    accr = _ACCR if dtype == jnp.float32 else _ACCB
    wps = accr // _NS

    @functools.partial(
        pl.kernel,
        out_type=jax.ShapeDtypeStruct((_NC, accr, dh), dtype),
        mesh=_mesh,
        scratch_types=[
            pltpu.VMEM((ichunk, _IDXW), jnp.int32),
            pltpu.VMEM((ichunk, _IDXW), jnp.int32),
            pltpu.VMEM((_IDXW, dh), dtype),
            pltpu.VMEM((_IDXW, dh), dtype),
            pltpu.VMEM_SHARED((accr, dh), dtype),
            pltpu.SemaphoreType.DMA,
            pltpu.SemaphoreType.DMA,
            pltpu.SemaphoreType.DMA,
            pltpu.SemaphoreType.DMA,
        ],
        compiler_params=cparams,
    )
    def agg(src3, dst2, table, zrows, out,
            src_v, dst_v, rows_a, rows_b, acc, sem_a, sem_b, ssem_a, ssem_b):
        c = lax.axis_index("c")
        w = lax.axis_index("s")
        pltpu.sync_copy(zrows, acc.at[pl.ds(w * wps, wps)])
        if feature_split:
            start = _FS_ROWS * w
        else:
            start = None
        plane = c
        plsc.subcore_barrier()

        bufs = (rows_a, rows_b)
        sems = (sem_a, sem_b)
        ssems = (ssem_a, ssem_b)

        def outer(k, carry):
            if feature_split:
                base = start + k * ichunk
            else:
                base = ((k * _NS + w) * _NC + c) * ichunk
            pltpu.sync_copy(src3.at[plane, pl.ds(base, ichunk)], src_v)
            pltpu.sync_copy(dst2.at[pl.ds(base, ichunk)], dst_v)
            descs = [None, None]
            sdescs = [None, None]
            descs[0] = pltpu.async_copy(table.at[src_v.at[0]], bufs[0],
                                        sems[0])
            for j in range(ichunk):
                p = j % 2
                descs[p].wait()
                sdescs[p] = pltpu.async_copy(bufs[p], acc.at[dst_v.at[j]],
                                             ssems[p], add=True)
                if j + 1 < ichunk:
                    q = (j + 1) % 2
                    if sdescs[q] is not None:
                        sdescs[q].wait()
                    descs[q] = pltpu.async_copy(
                        table.at[src_v.at[j + 1]], bufs[q], sems[q])
            sdescs[0].wait()
            sdescs[1].wait()
            return carry

        lax.fori_loop(0, nrows // ichunk, outer, 0)
        plsc.subcore_barrier()
        pltpu.sync_copy(acc.at[pl.ds(w * wps, wps)],
                        out.at[c, pl.ds(w * wps, wps)])

    return agg


_DEGW = 16

@functools.partial(
    pl.kernel,
    out_type=jax.ShapeDtypeStruct((_NC, _ACCR, _DEGW), jnp.float32),
    mesh=_mesh,
    scratch_types=[
        pltpu.VMEM((_ES_ROWS, _IDXW), jnp.int32),
        pltpu.VMEM((_IDXW, _DEGW), jnp.float32),
        pltpu.VMEM_SHARED((_ACCR, _DEGW), jnp.float32),
    ],
    compiler_params=pltpu.CompilerParams(use_tc_tiling_on_sc=False),
)
def _deg_kernel(dst2, zrows, ones, out, dst_v, ones_v, acc):
    c = lax.axis_index("c")
    w = lax.axis_index("s")
    pltpu.sync_copy(zrows, acc.at[pl.ds(w * _WPS, _WPS)])
    pltpu.sync_copy(ones, ones_v)
    start = (_NROW // _NC) * c + _ES_ROWS * w
    pltpu.sync_copy(dst2.at[pl.ds(start, _ES_ROWS)], dst_v)
    plsc.subcore_barrier()

    def body(j, carry):
        pltpu.sync_copy(ones_v, acc.at[dst_v.at[j]], add=True)
        return carry

    lax.fori_loop(0, _ES_ROWS, body, 0)
    plsc.subcore_barrier()
    pltpu.sync_copy(acc.at[pl.ds(w * _WPS, _WPS)],
                    out.at[c, pl.ds(w * _WPS, _WPS)])


_agg_fs = _make_agg(True)
_agg_es = _make_agg(False)
_agg_es64 = _make_agg(False, dh=64, tc_tiling=False)

_R = 1000


def _mlp_body(x_ref, degp_ref, w1, b1, w2, b2, w3, b3, hs_ref, dinv_ref):
    h = jnp.maximum(jnp.dot(x_ref[...], w1[...],
                            preferred_element_type=jnp.float32) + b1[...], 0.0)
    h = jnp.maximum(jnp.dot(h, w2[...],
                            preferred_element_type=jnp.float32) + b2[...], 0.0)
    h = jnp.maximum(jnp.dot(h, w3[...],
                            preferred_element_type=jnp.float32) + b3[...], 0.0)
    deg = 1.0 + degp_ref[0] + degp_ref[1]
    dinv = lax.rsqrt(deg)
    hs = h * dinv[:, 0:1]
    hs_ref[0] = hs
    hs_ref[1] = hs
    dinv_ref[...] = dinv


def _gcn1_body(s_ref, hs_ref, dinv_ref, w, b, out_ref):
    dinv = dinv_ref[:, 0:1]
    t = (s_ref[0] + s_ref[1] + hs_ref[0]) * dinv
    o = jnp.maximum(jnp.dot(t, w[...],
                            preferred_element_type=jnp.float32) + b[...], 0.0)
    hs2 = o * dinv
    out_ref[0] = hs2[:, :128]
    out_ref[1] = hs2[:, 128:]


def _gcn2_body(s_ref, hs_ref, dinv_ref, w2, b2, w3, out_ref):
    dinv = dinv_ref[:, 0:1]
    t = jnp.concatenate([s_ref[0] + hs_ref[0], s_ref[1] + hs_ref[1]],
                        axis=1) * dinv
    o = jnp.maximum(jnp.dot(t, w2[...],
                            preferred_element_type=jnp.float32) + b2[...], 0.0)
    g = jnp.dot(o, w3[...], preferred_element_type=jnp.float32)
    hs3 = jnp.concatenate(
        [g * dinv, jnp.zeros((_R, 14), jnp.float32)], axis=1)
    out_ref[0] = hs3
    out_ref[1] = hs3


def _final_body(s_ref, hs_ref, dinv_ref, b3, out_ref):
    t = (s_ref[0] + s_ref[1] + hs_ref[0]) * dinv_ref[:, 0:1]
    logits = t[:, :50] + b3[...]
    m = jnp.max(logits, axis=1, keepdims=True)
    e = jnp.exp(logits - m)
    out_ref[...] = e / jnp.sum(e, axis=1, keepdims=True)


def _full(shape):
    return pl.BlockSpec(shape, lambda i: tuple(0 for _ in shape))


def _rows(shape):
    return pl.BlockSpec(shape, lambda i: (i,) + tuple(0 for _ in shape[1:]))


def _planes(shape):
    return pl.BlockSpec(shape, lambda i: (0, i, 0))


def _plane0(shape):
    return pl.BlockSpec(shape, lambda i: (0, i, 0))


def kernel(x, edge_index, cnn1_W, cnn1_b, cnn2_W, cnn2_b, cnn3_W, cnn3_b,
           gcn1_W, gcn1_b, gcn2_W, gcn2_b, gcn3_W, gcn3_b):
    f32 = jnp.float32
    src = edge_index[0]
    dst = edge_index[1]
    npad = _NROW * _IDXW - _E
    src_p = jnp.concatenate([src, jnp.zeros((npad,), jnp.int32)])
    pad_dst = _N + jnp.arange(npad, dtype=jnp.int32) % (_ACCR - _N - 2)
    dst_p = jnp.concatenate([dst, pad_dst])
    src3 = jnp.stack([src_p, src_p + _N]).reshape(_NC, _NROW, _IDXW)
    dst2 = dst_p.reshape(_NROW, _IDXW)

    zrows = jnp.zeros((_WPS, _D), f32)
    zrows64 = jnp.zeros((_WPS, 64), f32)
    zrows16 = jnp.zeros((_WPS, _DEGW), f32)
    ones = jnp.ones((_IDXW, _DEGW), f32)

    degp = _deg_kernel(dst2, zrows16, ones)

    grid = (_N // _R,)
    hs1, dinv16 = pl.pallas_call(
        _mlp_body,
        grid=grid,
        in_specs=[
            _rows((_R, 3)), _planes((_NC, _R, _DEGW)),
            _full((3, 32)), _full((1, 32)),
            _full((32, 64)), _full((1, 64)),
            _full((64, 128)), _full((1, 128)),
        ],
        out_specs=[_planes((_NC, _R, 128)), _rows((_R, 16))],
        out_shape=[jax.ShapeDtypeStruct((_NC, _N, 128), f32),
                   jax.ShapeDtypeStruct((_N, 16), f32)],
    )(x, degp, cnn1_W, cnn1_b.reshape(1, 32), cnn2_W, cnn2_b.reshape(1, 64),
      cnn3_W, cnn3_b.reshape(1, 128))

    s1 = _agg_es(src3, dst2, hs1.reshape(_NC * _N, 128), zrows)

    hs2 = pl.pallas_call(
        _gcn1_body,
        grid=grid,
        in_specs=[
            _planes((_NC, _R, _D)), _plane0((1, _R, 128)), _rows((_R, 16)),
            _full((128, 256)), _full((1, 256)),
        ],
        out_specs=_planes((_NC, _R, 128)),
        out_shape=jax.ShapeDtypeStruct((_NC, _N, 128), f32),
    )(s1, hs1, dinv16, gcn1_W, gcn1_b.reshape(1, 256))

    s2 = _agg_fs(src3, dst2, hs2.reshape(_NC * _N, 128), zrows)

    hs3 = pl.pallas_call(
        _gcn2_body,
        grid=grid,
        in_specs=[
            _planes((_NC, _R, _D)), _planes((_NC, _R, 128)), _rows((_R, 16)),
            _full((256, 512)), _full((1, 512)), _full((512, 50)),
        ],
        out_specs=_planes((_NC, _R, 64)),
        out_shape=jax.ShapeDtypeStruct((_NC, _N, 64), f32),
    )(s2, hs2, dinv16, gcn2_W, gcn2_b.reshape(1, 512), gcn3_W)

    s3 = _agg_es64(src3, dst2, hs3.reshape(_NC * _N, 64), zrows64)

    out = pl.pallas_call(
        _final_body,
        grid=grid,
        in_specs=[
            _planes((_NC, _R, 64)), _plane0((1, _R, 64)), _rows((_R, 16)),
            _full((1, 50)),
        ],
        out_specs=_rows((_R, 50)),
        out_shape=jax.ShapeDtypeStruct((_N, 50), f32),
    )(s3, hs3, dinv16, gcn3_b.reshape(1, 50))

    return out
